# Initial kernel scaffold; baseline (speedup 1.0000x reference)
import jax
import jax.numpy as jnp
from jax.experimental import pallas as pl


def kernel(x, edge_index, batch, graph_features, enc_W, enc_b, enc_g, enc_beta, conv_W, conv_b, conv_g, conv_beta, fc1_W, fc1_b, bn1_g, bn1_beta, fc2_W, fc2_b, bn2_g, bn2_beta, fc3_W, fc3_b):
    raise NotImplementedError("write your pallas kernel here")


# speedup vs baseline: 9.0664x; 9.0664x over previous
import functools

import jax
import jax.numpy as jnp
from jax import lax
from jax.experimental import pallas as pl
from jax.experimental.pallas import tpu as pltpu
from jax.experimental.pallas import tpu_sc as plsc

_N, _E, _H = 10000, 320000, 128
_G, _NGF = 8, 32
_EPS = 1e-5
_INV = (1.0 + _EPS) ** -0.5

_NC, _NS = 2, 16
_NT = _NC * _NS
_EPT = -(-_E // _NT // 128) * 128
_NCH = _EPT // 128
_RPT = 624
_REM = _N - _NS * _RPT
_ACC_ROWS = _N + 8

_BR = 2000
_NB = _N // _BR


def _sc_degree(dst_r, ones_rows, zeros_deg):
    mesh = plsc.VectorSubcoreMesh(core_axis_name="c", subcore_axis_name="s")

    @functools.partial(
        pl.kernel,
        out_type=jax.ShapeDtypeStruct((_NC, _N, 128), jnp.float32),
        mesh=mesh,
        scratch_types=[
            pltpu.VMEM((_NCH, 128), jnp.int32),
            pltpu.VMEM((128, 128), jnp.float32),
            pltpu.VMEM_SHARED((_ACC_ROWS, 128), jnp.float32),
        ],
    )
    def k(dst_hbm, ones_hbm, zero_hbm, out_hbm, dst_v, ones_v, acc_sh):
        c = lax.axis_index("c")
        s = lax.axis_index("s")
        t = c * _NS + s
        base = s * _RPT
        pltpu.sync_copy(zero_hbm.at[pl.ds(base, _RPT)],
                        acc_sh.at[pl.ds(base, _RPT)])

        @pl.when(s == 0)
        def _():
            pltpu.sync_copy(zero_hbm.at[pl.ds(_NS * _RPT, _REM + 8)],
                            acc_sh.at[pl.ds(_NS * _RPT, _REM + 8)])

        pltpu.sync_copy(dst_hbm.at[t], dst_v)
        pltpu.sync_copy(ones_hbm, ones_v)
        plsc.subcore_barrier()

        def body(j, carry):
            pltpu.sync_copy(ones_v, acc_sh.at[dst_v.at[j]], add=True)
            return carry

        lax.fori_loop(0, _NCH, body, 0)
        plsc.subcore_barrier()
        pltpu.sync_copy(acc_sh.at[pl.ds(base, _RPT)],
                        out_hbm.at[c, pl.ds(base, _RPT)])

        @pl.when(s == 0)
        def _():
            pltpu.sync_copy(acc_sh.at[pl.ds(_NS * _RPT, _REM)],
                            out_hbm.at[c, pl.ds(_NS * _RPT, _REM)])

    return k(dst_r, ones_rows, zeros_deg)


def _sc_scatter(u, src_r, dst_r, zrow):
    mesh = plsc.VectorSubcoreMesh(core_axis_name="c", subcore_axis_name="s")

    @functools.partial(
        pl.kernel,
        out_type=jax.ShapeDtypeStruct((_NC, _N, _H), jnp.float32),
        mesh=mesh,
        scratch_types=[
            pltpu.VMEM((_NCH, 128), jnp.int32),
            pltpu.VMEM((_NCH, 128), jnp.int32),
            pltpu.VMEM((128, _H), jnp.float32),
            pltpu.VMEM_SHARED((_ACC_ROWS, _H), jnp.float32),
            pltpu.SemaphoreType.DMA,
        ],
    )
    def k(u_hbm, src_hbm, dst_hbm, zrow_hbm, out_hbm,
          src_v, dst_v, rows_v, acc_sh, sem):
        c = lax.axis_index("c")
        s = lax.axis_index("s")
        t = c * _NS + s
        base = s * _RPT
        pltpu.sync_copy(u_hbm.at[pl.ds(base, _RPT)],
                        acc_sh.at[pl.ds(base, _RPT)])

        @pl.when(s == 0)
        def _():
            pltpu.sync_copy(u_hbm.at[pl.ds(_NS * _RPT, _REM)],
                            acc_sh.at[pl.ds(_NS * _RPT, _REM)])
            pltpu.sync_copy(zrow_hbm, acc_sh.at[pl.ds(_N, 8)])

        pltpu.sync_copy(src_hbm.at[t], src_v)
        pltpu.sync_copy(dst_hbm.at[t], dst_v)
        plsc.subcore_barrier()

        def body(j, carry):
            pltpu.async_copy(u_hbm.at[src_v.at[j]], rows_v, sem).wait()
            pltpu.sync_copy(rows_v, acc_sh.at[dst_v.at[j]], add=True)
            return carry

        lax.fori_loop(0, _NCH, body, 0)
        plsc.subcore_barrier()
        pltpu.sync_copy(acc_sh.at[pl.ds(base, _RPT)],
                        out_hbm.at[c, pl.ds(base, _RPT)])

        @pl.when(s == 0)
        def _():
            pltpu.sync_copy(acc_sh.at[pl.ds(_NS * _RPT, _REM)],
                            out_hbm.at[c, pl.ds(_NS * _RPT, _REM)])

    return k(u, src_r, dst_r, zrow)


def _row_spec():
    return pl.BlockSpec((_BR, _H), lambda i: (i, 0))


def _full_spec(shape):
    return pl.BlockSpec(shape, lambda i: tuple(0 for _ in shape))


def _tc_encoder(x, W, b, g, beta):
    def body(x_ref, W_ref, b_ref, g_ref, beta_ref, h_ref):
        h = jnp.dot(x_ref[...], W_ref[...],
                    preferred_element_type=jnp.float32) + b_ref[...]
        h_ref[...] = jnp.maximum(g_ref[...] * _INV * h + beta_ref[...], 0.0)

    return pl.pallas_call(
        body,
        grid=(_NB,),
        in_specs=[
            pl.BlockSpec((_BR, x.shape[1]), lambda i: (i, 0)),
            _full_spec((x.shape[1], _H)),
            _full_spec((1, _H)),
            _full_spec((1, _H)),
            _full_spec((1, _H)),
        ],
        out_specs=_row_spec(),
        out_shape=jax.ShapeDtypeStruct((_N, _H), jnp.float32),
    )(x, W, b.reshape(1, _H), g.reshape(1, _H), beta.reshape(1, _H))


def _tc_first(h, deg0, deg1, W):
    def body(h_ref, d0_ref, d1_ref, W_ref, u_ref, dis_ref):
        deg = d0_ref[:, 0:1] + d1_ref[:, 0:1] + 1.0
        dis = lax.rsqrt(deg)
        u_ref[...] = jnp.dot(dis * h_ref[...], W_ref[...],
                             preferred_element_type=jnp.float32)
        dis_ref[...] = dis

    return pl.pallas_call(
        body,
        grid=(_NB,),
        in_specs=[
            _row_spec(),
            pl.BlockSpec((_BR, 128), lambda i: (i, 0)),
            pl.BlockSpec((_BR, 128), lambda i: (i, 0)),
            _full_spec((_H, _H)),
        ],
        out_specs=[_row_spec(), pl.BlockSpec((_BR, 1), lambda i: (i, 0))],
        out_shape=[
            jax.ShapeDtypeStruct((_N, _H), jnp.float32),
            jax.ShapeDtypeStruct((_N, 1), jnp.float32),
        ],
    )(h, deg0, deg1, W)


def _tc_mid(a0, a1, u_in, dis, b, g, beta, W_next, h_res=None, want_h=False):
    def body(*refs):
        idx = 8
        a0_ref, a1_ref, u_in_ref, dis_ref, b_ref, g_ref, beta_ref, W_ref = \
            refs[:8]
        if h_res is not None:
            res_ref = refs[idx]
            idx += 1
        u_ref = refs[idx]
        idx += 1
        d = dis_ref[...]
        hpre = d * (a0_ref[...] + a1_ref[...] - u_in_ref[...]) + b_ref[...]
        h = jnp.maximum(g_ref[...] * _INV * hpre + beta_ref[...], 0.0)
        if h_res is not None:
            h = h + res_ref[...]
        u_ref[...] = jnp.dot(d * h, W_ref[...],
                             preferred_element_type=jnp.float32)
        if want_h:
            refs[idx][...] = h

    in_specs = [
        _row_spec(), _row_spec(), _row_spec(),
        pl.BlockSpec((_BR, 1), lambda i: (i, 0)),
        _full_spec((1, _H)), _full_spec((1, _H)), _full_spec((1, _H)),
        _full_spec((_H, _H)),
    ]
    args = [a0, a1, u_in, dis, b.reshape(1, _H), g.reshape(1, _H),
            beta.reshape(1, _H), W_next]
    if h_res is not None:
        in_specs.append(_row_spec())
        args.append(h_res)
    out_specs = [_row_spec()]
    out_shape = [jax.ShapeDtypeStruct((_N, _H), jnp.float32)]
    if want_h:
        out_specs.append(_row_spec())
        out_shape.append(jax.ShapeDtypeStruct((_N, _H), jnp.float32))

    return pl.pallas_call(
        body,
        grid=(_NB,),
        in_specs=in_specs,
        out_specs=out_specs,
        out_shape=out_shape,
    )(*args)


def _tc_final(a0, a1, u_in, dis, b, g, beta, batch2d, gf,
              f1W, f1b, b1g, b1b, f2W, f2b, b2g, b2b, f3W, f3b):
    nin = 3 * _H + _NGF

    def body(a0_ref, a1_ref, u_in_ref, dis_ref, b_ref, g_ref, beta_ref,
             batch_ref,
             gf_ref, f1W_ref, f1b_ref, b1g_ref, b1b_ref, f2W_ref, f2b_ref,
             b2g_ref, b2b_ref, f3W_ref, f3b_ref, out_ref,
             ssum, smax, scnt):
        i = pl.program_id(0)

        @pl.when(i == 0)
        def _():
            ssum[...] = jnp.zeros_like(ssum)
            smax[...] = jnp.full_like(smax, -jnp.inf)
            scnt[...] = jnp.zeros_like(scnt)

        d = dis_ref[...]
        hpre = d * (a0_ref[...] + a1_ref[...] - u_in_ref[...]) + b_ref[...]
        h = jnp.maximum(g_ref[...] * _INV * hpre + beta_ref[...], 0.0)
        bb = batch_ref[...]
        for seg in range(_G):
            m = bb == seg
            hm = jnp.where(m, h, -jnp.inf)
            smax[seg:seg + 1, :] = jnp.maximum(
                smax[seg:seg + 1, :], jnp.max(hm, axis=0, keepdims=True))
            hs = jnp.where(m, h, 0.0)
            ssum[seg:seg + 1, :] += jnp.sum(hs, axis=0, keepdims=True)
            cnt = jnp.sum(jnp.where(m, 1.0, 0.0), axis=0, keepdims=True)
            scnt[seg:seg + 1, :] += cnt

        @pl.when(i == _NB - 1)
        def _():
            sums = ssum[...]
            mean = sums / jnp.maximum(scnt[...], 1.0)
            z = jnp.concatenate([mean, smax[...], sums, gf_ref[...]], axis=1)
            z = jnp.dot(z, f1W_ref[...],
                        preferred_element_type=jnp.float32) + f1b_ref[...]
            z = jnp.maximum(b1g_ref[...] * _INV * z + b1b_ref[...], 0.0)
            z = jnp.dot(z, f2W_ref[...],
                        preferred_element_type=jnp.float32) + f2b_ref[...]
            z = jnp.maximum(b2g_ref[...] * _INV * z + b2b_ref[...], 0.0)
            out_ref[...] = jnp.dot(
                z, f3W_ref[...],
                preferred_element_type=jnp.float32) + f3b_ref[...]

    c_out = f3W.shape[1]
    return pl.pallas_call(
        body,
        grid=(_NB,),
        in_specs=[
            _row_spec(), _row_spec(), _row_spec(),
            pl.BlockSpec((_BR, 1), lambda i: (i, 0)),
            _full_spec((1, _H)), _full_spec((1, _H)), _full_spec((1, _H)),
            pl.BlockSpec((_BR, 1), lambda i: (i, 0)),
            _full_spec((_G, _NGF)),
            _full_spec((nin, 2 * _H)), _full_spec((1, 2 * _H)),
            _full_spec((1, 2 * _H)), _full_spec((1, 2 * _H)),
            _full_spec((2 * _H, _H)), _full_spec((1, _H)),
            _full_spec((1, _H)), _full_spec((1, _H)),
            _full_spec((_H, c_out)), _full_spec((1, c_out)),
        ],
        out_specs=_full_spec((_G, c_out)),
        out_shape=jax.ShapeDtypeStruct((_G, c_out), jnp.float32),
        scratch_shapes=[
            pltpu.VMEM((_G, _H), jnp.float32),
            pltpu.VMEM((_G, _H), jnp.float32),
            pltpu.VMEM((_G, _H), jnp.float32),
        ],
    )(a0, a1, u_in, dis, b.reshape(1, _H), g.reshape(1, _H),
      beta.reshape(1, _H),
      batch2d, gf, f1W, f1b.reshape(1, 2 * _H), b1g.reshape(1, 2 * _H),
      b1b.reshape(1, 2 * _H), f2W, f2b.reshape(1, _H), b2g.reshape(1, _H),
      b2b.reshape(1, _H), f3W, f3b.reshape(1, c_out))


def kernel(x, edge_index, batch, graph_features, enc_W, enc_b, enc_g, enc_beta,
           conv_W, conv_b, conv_g, conv_beta, fc1_W, fc1_b, bn1_g, bn1_beta,
           fc2_W, fc2_b, bn2_g, bn2_beta, fc3_W, fc3_b):
    src = edge_index[0]
    dst = edge_index[1]
    pad = _NT * _EPT - _E
    src_r = jnp.concatenate(
        [src, jnp.zeros((pad,), jnp.int32)]).reshape(_NT, _NCH, 128)
    dst_r = jnp.concatenate(
        [dst, jnp.full((pad,), _N, jnp.int32)]).reshape(_NT, _NCH, 128)
    ones_rows = jnp.ones((128, 128), jnp.float32)
    zeros_deg = jnp.zeros((_ACC_ROWS, 128), jnp.float32)
    zrow = jnp.zeros((8, _H), jnp.float32)
    batch2d = batch.reshape(_N, 1)
    gf = graph_features.reshape(_G, _NGF)

    degp = _sc_degree(dst_r, ones_rows, zeros_deg)
    hA = _tc_encoder(x, enc_W, enc_b, enc_g, enc_beta)
    u0, dis = _tc_first(hA, degp[0], degp[1], conv_W[0])

    a = _sc_scatter(u0, src_r, dst_r, zrow)
    u1 = _tc_mid(a[0], a[1], u0, dis, conv_b[0], conv_g[0], conv_beta[0],
                 conv_W[1])[0]
    a = _sc_scatter(u1, src_r, dst_r, zrow)
    u2, h2 = _tc_mid(a[0], a[1], u1, dis, conv_b[1], conv_g[1], conv_beta[1],
                     conv_W[2], want_h=True)
    a = _sc_scatter(u2, src_r, dst_r, zrow)
    u3 = _tc_mid(a[0], a[1], u2, dis, conv_b[2], conv_g[2], conv_beta[2],
                 conv_W[3], h_res=h2)[0]
    a = _sc_scatter(u3, src_r, dst_r, zrow)
    return _tc_final(a[0], a[1], u3, dis, conv_b[3], conv_g[3], conv_beta[3],
                     batch2d, gf, fc1_W, fc1_b, bn1_g, bn1_beta,
                     fc2_W, fc2_b, bn2_g, bn2_beta, fc3_W, fc3_b)
